# baseline (speedup 1.0000x reference)
import functools

import jax
import jax.numpy as jnp
from jax import lax
from jax.experimental import pallas as pl
from jax.experimental.pallas import tpu as pltpu
from jax.experimental.pallas import tpu_sc as plsc

N = 10000
H = 128
OUT = 40
NC = 2
NS = 16
NW = NC * NS
CH = 128
N_PAD = 10240
ROWS = N_PAD // NS
DEG_W = 16
ZB = 64


NBUF = 2
SB = 8
DEGB = 4


def _fill(ref, nrows, val):
    @pl.loop(0, nrows)
    def _(i):
        @pl.loop(0, H, step=16)
        def _(j):
            ref[i, pl.ds(j, 16)] = jnp.full((16,), val, jnp.float32)


def _sc_agg_build(cpw, gather, solo=None):
    mesh = plsc.VectorSubcoreMesh(core_axis_name="c", subcore_axis_name="s")
    if gather:
        assert cpw % SB == 0
        nsb = cpw // SB
        scratch = [
            pltpu.VMEM((SB, CH), jnp.int32),
            pltpu.VMEM((SB, CH), jnp.int32),
            pltpu.VMEM((CH, H), jnp.float32),
            pltpu.VMEM((CH, H), jnp.float32),
            pltpu.VMEM_SHARED((N_PAD, H), jnp.float32),
            pltpu.SemaphoreType.DMA, pltpu.SemaphoreType.DMA,
            pltpu.SemaphoreType.DMA, pltpu.SemaphoreType.DMA,
        ]
    else:
        assert cpw % DEGB == 0
        scratch = [
            pltpu.VMEM((cpw, CH), jnp.int32),
            pltpu.VMEM((CH, H), jnp.float32),
            pltpu.VMEM_SHARED((N_PAD, H), jnp.float32),
        ] + [pltpu.SemaphoreType.DMA for _ in range(DEGB)]

    def body(refs):
        if gather:
            (h_hbm, src_hbm, dst_hbm, out_hbm,
             didx_b, sidx_b, r0, r1, acc, g0, g1, s0, s1) = refs
            rows, gsems, ssems = [r0, r1], [g0, g1], [s0, s1]
        else:
            dst_hbm, out_hbm, didx_all, ones_v, acc = refs[:5]
            ssems = list(refs[5:])
        c = lax.axis_index("c")
        s = lax.axis_index("s")
        wid = s * NC + c
        base = s * ROWS

        zbuf = rows[0] if gather else ones_v
        _fill(zbuf, CH, 0.0)

        @pl.loop(0, ROWS, step=CH)
        def _(r):
            pltpu.sync_copy(zbuf, acc.at[pl.ds(base + r, CH)])

        if not gather:
            _fill(ones_v, CH, 1.0)
            pltpu.sync_copy(dst_hbm.at[pl.ds(wid * cpw, cpw)], didx_all)

        plsc.subcore_barrier()

        if gather:
            eff_cpw = cpw * NC if solo is not None else cpw
            nsb_eff = eff_cpw // SB
            wid_g = s if solo is not None else wid

            def run_gather():
                @pl.loop(0, nsb_eff)
                def _(q):
                    @pl.when(q > 0)
                    def _():
                        for b in range(2):
                            pltpu.make_async_copy(
                                rows[b], acc.at[didx_b.at[SB - 2 + b]],
                                ssems[b]).wait()
                    qb = (wid_g * nsb_eff + q) * SB
                    pltpu.sync_copy(dst_hbm.at[pl.ds(qb, SB)], didx_b)
                    pltpu.sync_copy(src_hbm.at[pl.ds(qb, SB)], sidx_b)
                    gh = [None, None]
                    gh[0] = pltpu.async_copy(h_hbm.at[sidx_b.at[0]], rows[0],
                                             gsems[0])
                    for k in range(SB):
                        b = k % 2
                        nb = (k + 1) % 2
                        if k + 1 < SB:
                            if k >= 1:
                                pltpu.make_async_copy(
                                    rows[nb], acc.at[didx_b.at[k - 1]],
                                    ssems[nb]).wait()
                            gh[nb] = pltpu.async_copy(
                                h_hbm.at[sidx_b.at[k + 1]], rows[nb],
                                gsems[nb])
                        gh[b].wait()
                        pltpu.async_copy(rows[b], acc.at[didx_b.at[k]],
                                         ssems[b], add=True)

                for b in range(2):
                    pltpu.make_async_copy(
                        rows[b], acc.at[didx_b.at[SB - 2 + b]],
                        ssems[b]).wait()

            if solo is None:
                run_gather()
            else:
                @pl.when(c == solo)
                def _():
                    run_gather()
        else:
            @pl.loop(0, cpw, step=DEGB)
            def _(j):
                for b in range(DEGB):
                    @pl.when(j > 0)
                    def _(b=b):
                        pltpu.make_async_copy(
                            ones_v, acc.at[didx_all.at[j - DEGB + b]],
                            ssems[b]).wait()
                    pltpu.async_copy(ones_v, acc.at[didx_all.at[j + b]],
                                     ssems[b], add=True)

            for b in range(DEGB):
                pltpu.make_async_copy(
                    ones_v, acc.at[didx_all.at[cpw - DEGB + b]],
                    ssems[b]).wait()

        plsc.subcore_barrier()
        pltpu.sync_copy(acc.at[pl.ds(base, ROWS)],
                        out_hbm.at[pl.ds(c * N_PAD + base, ROWS)])

    out_type = jax.ShapeDtypeStruct((NC * N_PAD, H), jnp.float32)

    @functools.partial(pl.kernel, out_type=out_type, mesh=mesh,
                       scratch_types=scratch)
    def k(*refs):
        body(refs)

    return k


def _sc_degree(dst2d, cpw):
    return _sc_agg_build(cpw, gather=False)(dst2d)


def _sc_agg(h, src2d, dst2d, cpw, solo=None):
    return _sc_agg_build(cpw, gather=True, solo=solo)(h, src2d, dst2d)


CH2 = 32
NH = N // 2
NH_PAD = 5120
TRASH = 5100
ROWS_H = NH_PAD // NS


def _sc_agg_spmem_build(cpw2):
    SB2 = 4
    assert cpw2 % SB2 == 0
    nsb = cpw2 // SB2
    mesh = plsc.VectorSubcoreMesh(core_axis_name="c", subcore_axis_name="s")
    scratch = [
        pltpu.VMEM((SB2, CH2), jnp.int32),
        pltpu.VMEM((SB2, CH2), jnp.int32),
        pltpu.VMEM((CH2, H), jnp.float32),
        pltpu.VMEM((CH2, H), jnp.float32),
        pltpu.VMEM_SHARED((N, H), jnp.float32),
        pltpu.VMEM_SHARED((NH_PAD, H), jnp.float32),
        pltpu.SemaphoreType.DMA, pltpu.SemaphoreType.DMA,
        pltpu.SemaphoreType.DMA, pltpu.SemaphoreType.DMA,
    ]

    @functools.partial(
        pl.kernel,
        out_type=jax.ShapeDtypeStruct((NC * 2 * NH_PAD, H), jnp.float32),
        mesh=mesh, scratch_types=scratch)
    def k(h_hbm, src_hbm, dst_hbm, out_hbm,
          didx_b, sidx_b, r0, r1, h_sh, acc, g0, g1, s0, s1):
        rows, gsems, ssems = [r0, r1], [g0, g1], [s0, s1]
        c = lax.axis_index("c")
        s = lax.axis_index("s")
        wid = s * NC + c
        base = s * ROWS_H

        hr = (N // NS) // 8 * 8
        pltpu.sync_copy(h_hbm.at[pl.ds(s * hr, hr)], h_sh.at[pl.ds(s * hr, hr)])

        @pl.when(s == 0)
        def _():
            pltpu.sync_copy(h_hbm.at[pl.ds(NS * hr, N - NS * hr)],
                            h_sh.at[pl.ds(NS * hr, N - NS * hr)])

        _fill(rows[0], CH2, 0.0)

        for p in range(2):
            @pl.loop(0, ROWS_H, step=CH2)
            def _(r):
                pltpu.sync_copy(rows[0], acc.at[pl.ds(base + r, CH2)])

            plsc.subcore_barrier()

            lo = p * NH

            @pl.loop(0, nsb)
            def _(q):
                @pl.when(q > 0)
                def _():
                    for b in range(2):
                        pltpu.make_async_copy(
                            rows[b], acc.at[didx_b.at[SB2 - 2 + b]],
                            ssems[b]).wait()
                qb = (wid * nsb + q) * SB2
                pltpu.sync_copy(dst_hbm.at[pl.ds(qb, SB2)], didx_b)
                pltpu.sync_copy(src_hbm.at[pl.ds(qb, SB2)], sidx_b)
                for i in range(SB2):
                    for j in range(0, CH2, 16):
                        d = didx_b[i, pl.ds(j, 16)]
                        r = d - lo
                        ok = (r >= 0) & (r < NH)
                        didx_b[i, pl.ds(j, 16)] = jnp.where(ok, r, TRASH)
                gh = [None, None]
                gh[0] = pltpu.async_copy(h_sh.at[sidx_b.at[0]], rows[0],
                                         gsems[0])
                for kk in range(SB2):
                    b = kk % 2
                    nb = (kk + 1) % 2
                    if kk + 1 < SB2:
                        if kk >= 1:
                            pltpu.make_async_copy(
                                rows[nb], acc.at[didx_b.at[kk - 1]],
                                ssems[nb]).wait()
                        gh[nb] = pltpu.async_copy(
                            h_sh.at[sidx_b.at[kk + 1]], rows[nb], gsems[nb])
                    gh[b].wait()
                    pltpu.async_copy(rows[b], acc.at[didx_b.at[kk]],
                                     ssems[b], add=True)

            for b in range(2):
                pltpu.make_async_copy(rows[b], acc.at[didx_b.at[SB2 - 2 + b]],
                                      ssems[b]).wait()

            plsc.subcore_barrier()
            pltpu.sync_copy(
                acc.at[pl.ds(base, ROWS_H)],
                out_hbm.at[pl.ds(c * 2 * NH_PAD + p * NH_PAD + base, ROWS_H)])
            if p == 0:
                _fill(rows[0], CH2, 0.0)

    return k


CH3 = 16
SB3 = 16
NP3 = 10112
ROWS3 = NP3 // NS
RH = 4400
SPZ = RH
HZ = N


def _sc_agg_hybrid_build(cpw3):
    assert cpw3 % SB3 == 0
    nsb = cpw3 // SB3
    mesh = plsc.VectorSubcoreMesh(core_axis_name="c", subcore_axis_name="s")
    scratch = [
        pltpu.VMEM((3 * SB3, CH3), jnp.int32),
        pltpu.VMEM((CH3, H), jnp.float32),
        pltpu.VMEM((CH3, H), jnp.float32),
        pltpu.VMEM((CH3, H), jnp.float32),
        pltpu.VMEM((CH3, H), jnp.float32),
        pltpu.VMEM_SHARED((RH + 8, H), jnp.float32),
        pltpu.VMEM_SHARED((NP3, H), jnp.float32),
    ] + [pltpu.SemaphoreType.DMA] * 8

    @functools.partial(
        pl.kernel,
        out_type=jax.ShapeDtypeStruct((NC * NP3, H), jnp.float32),
        mesh=mesh, scratch_types=scratch)
    def k(ha_hbm, src_hbm, dst_hbm, out_hbm,
          idx_a, l0, l1, h0, h1, h_sh, acc, *sems):
        bufL, bufH = [l0, l1], [h0, h1]
        gsL, gsH, ssL, ssH = ([sems[0], sems[1]], [sems[2], sems[3]],
                              [sems[4], sems[5]], [sems[6], sems[7]])
        c = lax.axis_index("c")
        s = lax.axis_index("s")
        wid = s * NC + c
        base = s * ROWS3
        c5 = c * RH

        def D(i):
            return idx_a.at[i]

        def L(i):
            return idx_a.at[SB3 + i]

        def Hx(i):
            return idx_a.at[2 * SB3 + i]

        _fill(bufL[0], CH3, 0.0)

        hr = (RH // NS) // 8 * 8
        pltpu.sync_copy(ha_hbm.at[pl.ds(c5 + s * hr, hr)],
                        h_sh.at[pl.ds(s * hr, hr)])

        @pl.when(s == 0)
        def _():
            pltpu.sync_copy(ha_hbm.at[pl.ds(c5 + NS * hr, RH - NS * hr)],
                            h_sh.at[pl.ds(NS * hr, RH - NS * hr)])

        @pl.when(s == 1)
        def _():
            pltpu.sync_copy(bufL[0].at[pl.ds(0, 8)], h_sh.at[pl.ds(RH, 8)])

        @pl.loop(0, ROWS3 // CH3 * CH3, step=CH3)
        def _(r):
            pltpu.sync_copy(bufL[0], acc.at[pl.ds(base + r, CH3)])

        pltpu.sync_copy(bufL[0].at[pl.ds(0, ROWS3 % CH3)],
                        acc.at[pl.ds(base + ROWS3 // CH3 * CH3,
                                     ROWS3 % CH3)])

        plsc.subcore_barrier()

        @pl.loop(0, nsb)
        def _(q):
            @pl.when(q > 0)
            def _():
                for b in range(2):
                    kk = SB3 - 2 + b
                    pltpu.make_async_copy(bufL[b], acc.at[D(kk)],
                                          ssL[b]).wait()
                    pltpu.make_async_copy(bufH[b], acc.at[D(kk)],
                                          ssH[b]).wait()
            qb = (wid * nsb + q) * SB3
            pltpu.sync_copy(dst_hbm.at[pl.ds(qb, SB3)],
                            idx_a.at[pl.ds(0, SB3)])
            pltpu.sync_copy(src_hbm.at[pl.ds(qb, SB3)],
                            idx_a.at[pl.ds(SB3, SB3)])
            for i in range(SB3):
                sv = idx_a[SB3 + i, pl.ds(0, CH3)]
                loc = (sv >= c5) & (sv < c5 + RH)
                idx_a[SB3 + i, pl.ds(0, CH3)] = jnp.where(loc, sv - c5, SPZ)
                idx_a[2 * SB3 + i, pl.ds(0, CH3)] = jnp.where(loc, HZ, sv)

            def g_start(kk, b):
                return (pltpu.async_copy(h_sh.at[L(kk)], bufL[b],
                                         gsL[b]),
                        pltpu.async_copy(ha_hbm.at[Hx(kk)], bufH[b],
                                         gsH[b]))

            gh = [None, None]
            gh[0] = g_start(0, 0)
            for kk in range(SB3):
                b = kk % 2
                nb = (kk + 1) % 2
                if kk + 1 < SB3:
                    if kk >= 1:
                        pltpu.make_async_copy(
                            bufL[nb], acc.at[D(kk - 1)],
                            ssL[nb]).wait()
                        pltpu.make_async_copy(
                            bufH[nb], acc.at[D(kk - 1)],
                            ssH[nb]).wait()
                    gh[nb] = g_start(kk + 1, nb)
                gh[b][0].wait()
                gh[b][1].wait()
                pltpu.async_copy(bufL[b], acc.at[D(kk)],
                                 ssL[b], add=True)
                pltpu.async_copy(bufH[b], acc.at[D(kk)],
                                 ssH[b], add=True)

        for b in range(2):
            kk = SB3 - 2 + b
            pltpu.make_async_copy(bufL[b], acc.at[D(kk)],
                                  ssL[b]).wait()
            pltpu.make_async_copy(bufH[b], acc.at[D(kk)],
                                  ssH[b]).wait()

        plsc.subcore_barrier()
        pltpu.sync_copy(acc.at[pl.ds(base, ROWS3)],
                        out_hbm.at[pl.ds(c * NP3 + base, ROWS3)])

    return k


BR = 400


def _tc_layer_body(a0, a1, d0, d1, h_ref, wl, blr, wr, o_ref):
    cnt = d0[:, 0:1] + d1[:, 0:1]
    inv = 1.0 / jnp.maximum(cnt, 1.0)
    mean = (a0[...] + a1[...]) * inv
    acc = lax.dot_general(mean, wl[...], (((1,), (1,)), ((), ())),
                          preferred_element_type=jnp.float32,
                          precision=lax.Precision.HIGHEST)
    acc = acc + blr[...]
    acc = acc + lax.dot_general(h_ref[...], wr[...], (((1,), (1,)), ((), ())),
                                preferred_element_type=jnp.float32,
                                precision=lax.Precision.HIGHEST)
    o_ref[...] = jnp.maximum(acc, 0.0)


def _tc_layer(a0, a1, d0, d1, h, Wl, bl, Wr):
    nb = N // BR
    return pl.pallas_call(
        _tc_layer_body,
        grid=(nb,),
        in_specs=[
            pl.BlockSpec((BR, H), lambda i: (i, 0)),
            pl.BlockSpec((BR, H), lambda i: (i, 0)),
            pl.BlockSpec((BR, H), lambda i: (i, 0)),
            pl.BlockSpec((BR, H), lambda i: (i, 0)),
            pl.BlockSpec((BR, H), lambda i: (i, 0)),
            pl.BlockSpec((H, H), lambda i: (0, 0)),
            pl.BlockSpec((1, H), lambda i: (0, 0)),
            pl.BlockSpec((H, H), lambda i: (0, 0)),
        ],
        out_specs=pl.BlockSpec((BR, H), lambda i: (i, 0)),
        out_shape=jax.ShapeDtypeStruct((N, H), jnp.float32),
    )(a0, a1, d0, d1, h, Wl, bl, Wr)


def _tc_final_body(h1, h2, h3, w1, w2, w3, br, o_ref):
    acc = lax.dot_general(h1[...], w1[...], (((1,), (1,)), ((), ())),
                          preferred_element_type=jnp.float32,
                          precision=lax.Precision.HIGHEST)
    acc = acc + lax.dot_general(h2[...], w2[...], (((1,), (1,)), ((), ())),
                                preferred_element_type=jnp.float32,
                                precision=lax.Precision.HIGHEST)
    acc = acc + lax.dot_general(h3[...], w3[...], (((1,), (1,)), ((), ())),
                                preferred_element_type=jnp.float32,
                                precision=lax.Precision.HIGHEST)
    o_ref[...] = acc + br[...]


def _tc_final(h1, h2, h3, w1, w2, w3, fc_b):
    nb = N // BR
    return pl.pallas_call(
        _tc_final_body,
        grid=(nb,),
        in_specs=[
            pl.BlockSpec((BR, H), lambda i: (i, 0)),
            pl.BlockSpec((BR, H), lambda i: (i, 0)),
            pl.BlockSpec((BR, H), lambda i: (i, 0)),
            pl.BlockSpec((OUT, H), lambda i: (0, 0)),
            pl.BlockSpec((OUT, H), lambda i: (0, 0)),
            pl.BlockSpec((OUT, H), lambda i: (0, 0)),
            pl.BlockSpec((1, OUT), lambda i: (0, 0)),
        ],
        out_specs=pl.BlockSpec((BR, OUT), lambda i: (i, 0)),
        out_shape=jax.ShapeDtypeStruct((N, OUT), jnp.float32),
    )(h1, h2, h3, w1, w2, w3, fc_b)


def kernel(x, edge_index, Wl0, bl0, Wr0, Wl1, bl1, Wr1, Wl2, bl2, Wr2, fc_W, fc_b):
    src = edge_index[0]
    dst = edge_index[1]
    e = src.shape[0]
    cpw = -(-e // (NW * CH))
    cpw = -(-cpw // SB) * SB
    e_pad = NW * CH * cpw
    src_p = jnp.concatenate(
        [src, jnp.zeros((e_pad - e,), jnp.int32)]).reshape(-1, CH)
    dst_p = jnp.concatenate(
        [dst, jnp.full((e_pad - e,), N, jnp.int32)]).reshape(-1, CH)

    degs = _sc_degree(dst_p, cpw)
    d0 = degs[0:N]
    d1 = degs[N_PAD:N_PAD + N]

    src_p3 = src_p.reshape(-1, CH3)
    dst_p3 = dst_p.reshape(-1, CH3)
    cpw3 = src_p3.shape[0] // NW
    agg3 = _sc_agg_hybrid_build(cpw3)
    zrows = jnp.zeros((8, H), jnp.float32)

    h = x
    hs = []
    for (Wl, bl, Wr) in ((Wl0, bl0, Wr0), (Wl1, bl1, Wr1), (Wl2, bl2, Wr2)):
        ha = jnp.concatenate([h, zrows])
        parts = agg3(ha, src_p3, dst_p3)
        h = _tc_layer(parts[0:N], parts[NP3:NP3 + N], d0, d1, h,
                      Wl, bl.reshape(1, H), Wr)
        hs.append(h)

    return _tc_final(hs[0], hs[1], hs[2],
                     fc_W[:, 0:H], fc_W[:, H:2 * H], fc_W[:, 2 * H:3 * H],
                     fc_b.reshape(1, OUT))


# speedup vs baseline: 8.5328x; 8.5328x over previous
import functools

import jax
import jax.numpy as jnp
from jax import lax
from jax.experimental import pallas as pl
from jax.experimental.pallas import tpu as pltpu
from jax.experimental.pallas import tpu_sc as plsc

N = 10000
H = 128
OUT = 40
NC = 2
NS = 16
NW = NC * NS
CH = 128
N_PAD = 10240
ROWS = N_PAD // NS
DEGB = 4


def _fill(ref, nrows, val):
    @pl.loop(0, nrows)
    def _(i):
        @pl.loop(0, H, step=16)
        def _(j):
            ref[i, pl.ds(j, 16)] = jnp.full((16,), val, jnp.float32)


def _sc_agg_build(cpw, gather):
    mesh = plsc.VectorSubcoreMesh(core_axis_name="c", subcore_axis_name="s")
    if gather:
        scratch = [
            pltpu.VMEM((1, CH), jnp.int32),
            pltpu.VMEM((1, CH), jnp.int32),
            pltpu.VMEM((CH, H), jnp.float32),
            pltpu.VMEM_SHARED((N_PAD, H), jnp.float32),
            pltpu.SemaphoreType.DMA,
        ]
    else:
        assert cpw % DEGB == 0
        scratch = [
            pltpu.VMEM((cpw, CH), jnp.int32),
            pltpu.VMEM((CH, H), jnp.float32),
            pltpu.VMEM_SHARED((N_PAD, H), jnp.float32),
        ] + [pltpu.SemaphoreType.DMA for _ in range(DEGB)]

    def body(refs):
        if gather:
            h_hbm, src_hbm, dst_hbm, out_hbm, sidx, didx, rows_v, acc, sem = refs
            zbuf = rows_v
        else:
            dst_hbm, out_hbm, didx_all, ones_v, acc = refs[:5]
            ssems = list(refs[5:])
            zbuf = ones_v
        c = lax.axis_index("c")
        s = lax.axis_index("s")
        wid = s * NC + c
        base = s * ROWS

        _fill(zbuf, CH, 0.0)

        @pl.loop(0, ROWS, step=CH)
        def _(r):
            pltpu.sync_copy(zbuf, acc.at[pl.ds(base + r, CH)])

        if not gather:
            _fill(ones_v, CH, 1.0)
            pltpu.sync_copy(dst_hbm.at[pl.ds(wid * cpw, cpw)], didx_all)

        plsc.subcore_barrier()

        if gather:
            @pl.loop(0, cpw)
            def _(j):
                pltpu.sync_copy(src_hbm.at[pl.ds(wid * cpw + j, 1)], sidx)
                pltpu.sync_copy(dst_hbm.at[pl.ds(wid * cpw + j, 1)], didx)
                pltpu.async_copy(h_hbm.at[sidx.at[0]], rows_v, sem).wait()
                pltpu.sync_copy(rows_v, acc.at[didx.at[0]], add=True)
        else:
            @pl.loop(0, cpw, step=DEGB)
            def _(j):
                for b in range(DEGB):
                    @pl.when(j > 0)
                    def _(b=b):
                        pltpu.make_async_copy(
                            ones_v, acc.at[didx_all.at[j - DEGB + b]],
                            ssems[b]).wait()
                    pltpu.async_copy(ones_v, acc.at[didx_all.at[j + b]],
                                     ssems[b], add=True)

            for b in range(DEGB):
                pltpu.make_async_copy(
                    ones_v, acc.at[didx_all.at[cpw - DEGB + b]],
                    ssems[b]).wait()

        plsc.subcore_barrier()
        pltpu.sync_copy(acc.at[pl.ds(base, ROWS)],
                        out_hbm.at[pl.ds(c * N_PAD + base, ROWS)])

    out_type = jax.ShapeDtypeStruct((NC * N_PAD, H), jnp.float32)

    @functools.partial(pl.kernel, out_type=out_type, mesh=mesh,
                       scratch_types=scratch)
    def k(*refs):
        body(refs)

    return k


def _sc_degree(dst2d, cpw):
    return _sc_agg_build(cpw, gather=False)(dst2d)


def _sc_agg(h, src2d, dst2d, cpw):
    return _sc_agg_build(cpw, gather=True)(h, src2d, dst2d)


BR = 400


def _tc_layer_body(a0, a1, d0, d1, h_ref, wl, blr, wr, o_ref):
    cnt = d0[:, 0:1] + d1[:, 0:1]
    inv = 1.0 / jnp.maximum(cnt, 1.0)
    mean = (a0[...] + a1[...]) * inv
    acc = lax.dot_general(mean, wl[...], (((1,), (1,)), ((), ())),
                          preferred_element_type=jnp.float32,
                          precision=lax.Precision.HIGHEST)
    acc = acc + blr[...]
    acc = acc + lax.dot_general(h_ref[...], wr[...], (((1,), (1,)), ((), ())),
                                preferred_element_type=jnp.float32,
                                precision=lax.Precision.HIGHEST)
    o_ref[...] = jnp.maximum(acc, 0.0)


def _tc_layer(a0, a1, d0, d1, h, Wl, bl, Wr):
    nb = N // BR
    return pl.pallas_call(
        _tc_layer_body,
        grid=(nb,),
        in_specs=[
            pl.BlockSpec((BR, H), lambda i: (i, 0)),
            pl.BlockSpec((BR, H), lambda i: (i, 0)),
            pl.BlockSpec((BR, H), lambda i: (i, 0)),
            pl.BlockSpec((BR, H), lambda i: (i, 0)),
            pl.BlockSpec((BR, H), lambda i: (i, 0)),
            pl.BlockSpec((H, H), lambda i: (0, 0)),
            pl.BlockSpec((1, H), lambda i: (0, 0)),
            pl.BlockSpec((H, H), lambda i: (0, 0)),
        ],
        out_specs=pl.BlockSpec((BR, H), lambda i: (i, 0)),
        out_shape=jax.ShapeDtypeStruct((N, H), jnp.float32),
    )(a0, a1, d0, d1, h, Wl, bl, Wr)


def _tc_final_body(h1, h2, h3, w1, w2, w3, br, o_ref):
    acc = lax.dot_general(h1[...], w1[...], (((1,), (1,)), ((), ())),
                          preferred_element_type=jnp.float32,
                          precision=lax.Precision.HIGHEST)
    acc = acc + lax.dot_general(h2[...], w2[...], (((1,), (1,)), ((), ())),
                                preferred_element_type=jnp.float32,
                                precision=lax.Precision.HIGHEST)
    acc = acc + lax.dot_general(h3[...], w3[...], (((1,), (1,)), ((), ())),
                                preferred_element_type=jnp.float32,
                                precision=lax.Precision.HIGHEST)
    o_ref[...] = acc + br[...]


def _tc_final(h1, h2, h3, w1, w2, w3, fc_b):
    nb = N // BR
    return pl.pallas_call(
        _tc_final_body,
        grid=(nb,),
        in_specs=[
            pl.BlockSpec((BR, H), lambda i: (i, 0)),
            pl.BlockSpec((BR, H), lambda i: (i, 0)),
            pl.BlockSpec((BR, H), lambda i: (i, 0)),
            pl.BlockSpec((OUT, H), lambda i: (0, 0)),
            pl.BlockSpec((OUT, H), lambda i: (0, 0)),
            pl.BlockSpec((OUT, H), lambda i: (0, 0)),
            pl.BlockSpec((1, OUT), lambda i: (0, 0)),
        ],
        out_specs=pl.BlockSpec((BR, OUT), lambda i: (i, 0)),
        out_shape=jax.ShapeDtypeStruct((N, OUT), jnp.float32),
    )(h1, h2, h3, w1, w2, w3, fc_b)


def kernel(x, edge_index, Wl0, bl0, Wr0, Wl1, bl1, Wr1, Wl2, bl2, Wr2, fc_W, fc_b):
    src = edge_index[0]
    dst = edge_index[1]
    e = src.shape[0]
    cpw = -(-e // (NW * CH))
    cpw = -(-cpw // DEGB) * DEGB
    e_pad = NW * CH * cpw
    src_p = jnp.concatenate(
        [src, jnp.zeros((e_pad - e,), jnp.int32)]).reshape(-1, CH)
    dst_p = jnp.concatenate(
        [dst, jnp.full((e_pad - e,), N, jnp.int32)]).reshape(-1, CH)

    degs = _sc_degree(dst_p, cpw)
    d0 = degs[0:N]
    d1 = degs[N_PAD:N_PAD + N]

    h = x
    hs = []
    for (Wl, bl, Wr) in ((Wl0, bl0, Wr0), (Wl1, bl1, Wr1), (Wl2, bl2, Wr2)):
        parts = _sc_agg(h, src_p, dst_p, cpw)
        h = _tc_layer(parts[0:N], parts[N_PAD:N_PAD + N], d0, d1, h,
                      Wl, bl.reshape(1, H), Wr)
        hs.append(h)

    return _tc_final(hs[0], hs[1], hs[2],
                     fc_W[:, 0:H], fc_W[:, H:2 * H], fc_W[:, 2 * H:3 * H],
                     fc_b.reshape(1, OUT))


# speedup vs baseline: 9.7299x; 1.1403x over previous
import functools

import jax
import jax.numpy as jnp
from jax import lax
from jax.experimental import pallas as pl
from jax.experimental.pallas import tpu as pltpu
from jax.experimental.pallas import tpu_sc as plsc

N = 10000
H = 128
OUT = 40
NC = 2
NS = 16
NW = NC * NS
CH = 128
N_PAD = 10240
ROWS = N_PAD // NS
DEG_W = 16
ZB = 64


NBUF = 2
SB = 8
DEGB = 4


def _fill(ref, nrows, val):
    @pl.loop(0, nrows)
    def _(i):
        @pl.loop(0, H, step=16)
        def _(j):
            ref[i, pl.ds(j, 16)] = jnp.full((16,), val, jnp.float32)


def _sc_agg_build(cpw, gather, solo=None):
    mesh = plsc.VectorSubcoreMesh(core_axis_name="c", subcore_axis_name="s")
    if gather:
        assert cpw % SB == 0
        nsb = cpw // SB
        scratch = [
            pltpu.VMEM((SB, CH), jnp.int32),
            pltpu.VMEM((SB, CH), jnp.int32),
            pltpu.VMEM((CH, H), jnp.float32),
            pltpu.VMEM((CH, H), jnp.float32),
            pltpu.VMEM_SHARED((N_PAD, H), jnp.float32),
            pltpu.SemaphoreType.DMA, pltpu.SemaphoreType.DMA,
            pltpu.SemaphoreType.DMA, pltpu.SemaphoreType.DMA,
        ]
    else:
        assert cpw % DEGB == 0
        scratch = [
            pltpu.VMEM((cpw, CH), jnp.int32),
            pltpu.VMEM((CH, H), jnp.float32),
            pltpu.VMEM_SHARED((N_PAD, H), jnp.float32),
        ] + [pltpu.SemaphoreType.DMA for _ in range(DEGB)]

    def body(refs):
        if gather:
            (h_hbm, src_hbm, dst_hbm, out_hbm,
             didx_b, sidx_b, r0, r1, acc, g0, g1, s0, s1) = refs
            rows, gsems, ssems = [r0, r1], [g0, g1], [s0, s1]
        else:
            dst_hbm, out_hbm, didx_all, ones_v, acc = refs[:5]
            ssems = list(refs[5:])
        c = lax.axis_index("c")
        s = lax.axis_index("s")
        wid = s * NC + c
        base = s * ROWS

        zbuf = rows[0] if gather else ones_v
        _fill(zbuf, CH, 0.0)

        @pl.loop(0, ROWS, step=CH)
        def _(r):
            pltpu.sync_copy(zbuf, acc.at[pl.ds(base + r, CH)])

        if not gather:
            _fill(ones_v, CH, 1.0)
            pltpu.sync_copy(dst_hbm.at[pl.ds(wid * cpw, cpw)], didx_all)

        plsc.subcore_barrier()

        if gather:
            eff_cpw = cpw * NC if solo is not None else cpw
            nsb_eff = eff_cpw // SB
            wid_g = s if solo is not None else wid

            def run_gather():
                @pl.loop(0, nsb_eff)
                def _(q):
                    @pl.when(q > 0)
                    def _():
                        for b in range(2):
                            pltpu.make_async_copy(
                                rows[b], acc.at[didx_b.at[SB - 2 + b]],
                                ssems[b]).wait()
                    qb = (wid_g * nsb_eff + q) * SB
                    pltpu.sync_copy(dst_hbm.at[pl.ds(qb, SB)], didx_b)
                    pltpu.sync_copy(src_hbm.at[pl.ds(qb, SB)], sidx_b)
                    gh = [None, None]
                    gh[0] = pltpu.async_copy(h_hbm.at[sidx_b.at[0]], rows[0],
                                             gsems[0])
                    for k in range(SB):
                        b = k % 2
                        nb = (k + 1) % 2
                        if k + 1 < SB:
                            if k >= 1:
                                pltpu.make_async_copy(
                                    rows[nb], acc.at[didx_b.at[k - 1]],
                                    ssems[nb]).wait()
                            gh[nb] = pltpu.async_copy(
                                h_hbm.at[sidx_b.at[k + 1]], rows[nb],
                                gsems[nb])
                        gh[b].wait()
                        pltpu.async_copy(rows[b], acc.at[didx_b.at[k]],
                                         ssems[b], add=True)

                for b in range(2):
                    pltpu.make_async_copy(
                        rows[b], acc.at[didx_b.at[SB - 2 + b]],
                        ssems[b]).wait()

            if solo is None:
                run_gather()
            else:
                @pl.when(c == solo)
                def _():
                    run_gather()
        else:
            @pl.loop(0, cpw, step=DEGB)
            def _(j):
                for b in range(DEGB):
                    @pl.when(j > 0)
                    def _(b=b):
                        pltpu.make_async_copy(
                            ones_v, acc.at[didx_all.at[j - DEGB + b]],
                            ssems[b]).wait()
                    pltpu.async_copy(ones_v, acc.at[didx_all.at[j + b]],
                                     ssems[b], add=True)

            for b in range(DEGB):
                pltpu.make_async_copy(
                    ones_v, acc.at[didx_all.at[cpw - DEGB + b]],
                    ssems[b]).wait()

        plsc.subcore_barrier()
        pltpu.sync_copy(acc.at[pl.ds(base, ROWS)],
                        out_hbm.at[pl.ds(c * N_PAD + base, ROWS)])

    out_type = jax.ShapeDtypeStruct((NC * N_PAD, H), jnp.float32)

    @functools.partial(pl.kernel, out_type=out_type, mesh=mesh,
                       scratch_types=scratch)
    def k(*refs):
        body(refs)

    return k


def _sc_degree(dst2d, cpw):
    return _sc_agg_build(cpw, gather=False)(dst2d)


def _sc_agg(h, src2d, dst2d, cpw, solo=None):
    return _sc_agg_build(cpw, gather=True, solo=solo)(h, src2d, dst2d)


CH2 = 32
NH = N // 2
NH_PAD = 5120
TRASH = 5100
ROWS_H = NH_PAD // NS


def _sc_agg_spmem_build(cpw2):
    SB2 = 4
    assert cpw2 % SB2 == 0
    nsb = cpw2 // SB2
    mesh = plsc.VectorSubcoreMesh(core_axis_name="c", subcore_axis_name="s")
    scratch = [
        pltpu.VMEM((SB2, CH2), jnp.int32),
        pltpu.VMEM((SB2, CH2), jnp.int32),
        pltpu.VMEM((CH2, H), jnp.float32),
        pltpu.VMEM((CH2, H), jnp.float32),
        pltpu.VMEM_SHARED((N, H), jnp.float32),
        pltpu.VMEM_SHARED((NH_PAD, H), jnp.float32),
        pltpu.SemaphoreType.DMA, pltpu.SemaphoreType.DMA,
        pltpu.SemaphoreType.DMA, pltpu.SemaphoreType.DMA,
    ]

    @functools.partial(
        pl.kernel,
        out_type=jax.ShapeDtypeStruct((NC * 2 * NH_PAD, H), jnp.float32),
        mesh=mesh, scratch_types=scratch)
    def k(h_hbm, src_hbm, dst_hbm, out_hbm,
          didx_b, sidx_b, r0, r1, h_sh, acc, g0, g1, s0, s1):
        rows, gsems, ssems = [r0, r1], [g0, g1], [s0, s1]
        c = lax.axis_index("c")
        s = lax.axis_index("s")
        wid = s * NC + c
        base = s * ROWS_H

        hr = (N // NS) // 8 * 8
        pltpu.sync_copy(h_hbm.at[pl.ds(s * hr, hr)], h_sh.at[pl.ds(s * hr, hr)])

        @pl.when(s == 0)
        def _():
            pltpu.sync_copy(h_hbm.at[pl.ds(NS * hr, N - NS * hr)],
                            h_sh.at[pl.ds(NS * hr, N - NS * hr)])

        _fill(rows[0], CH2, 0.0)

        for p in range(2):
            @pl.loop(0, ROWS_H, step=CH2)
            def _(r):
                pltpu.sync_copy(rows[0], acc.at[pl.ds(base + r, CH2)])

            plsc.subcore_barrier()

            lo = p * NH

            @pl.loop(0, nsb)
            def _(q):
                @pl.when(q > 0)
                def _():
                    for b in range(2):
                        pltpu.make_async_copy(
                            rows[b], acc.at[didx_b.at[SB2 - 2 + b]],
                            ssems[b]).wait()
                qb = (wid * nsb + q) * SB2
                pltpu.sync_copy(dst_hbm.at[pl.ds(qb, SB2)], didx_b)
                pltpu.sync_copy(src_hbm.at[pl.ds(qb, SB2)], sidx_b)
                for i in range(SB2):
                    for j in range(0, CH2, 16):
                        d = didx_b[i, pl.ds(j, 16)]
                        r = d - lo
                        ok = (r >= 0) & (r < NH)
                        didx_b[i, pl.ds(j, 16)] = jnp.where(ok, r, TRASH)
                gh = [None, None]
                gh[0] = pltpu.async_copy(h_sh.at[sidx_b.at[0]], rows[0],
                                         gsems[0])
                for kk in range(SB2):
                    b = kk % 2
                    nb = (kk + 1) % 2
                    if kk + 1 < SB2:
                        if kk >= 1:
                            pltpu.make_async_copy(
                                rows[nb], acc.at[didx_b.at[kk - 1]],
                                ssems[nb]).wait()
                        gh[nb] = pltpu.async_copy(
                            h_sh.at[sidx_b.at[kk + 1]], rows[nb], gsems[nb])
                    gh[b].wait()
                    pltpu.async_copy(rows[b], acc.at[didx_b.at[kk]],
                                     ssems[b], add=True)

            for b in range(2):
                pltpu.make_async_copy(rows[b], acc.at[didx_b.at[SB2 - 2 + b]],
                                      ssems[b]).wait()

            plsc.subcore_barrier()
            pltpu.sync_copy(
                acc.at[pl.ds(base, ROWS_H)],
                out_hbm.at[pl.ds(c * 2 * NH_PAD + p * NH_PAD + base, ROWS_H)])
            if p == 0:
                _fill(rows[0], CH2, 0.0)

    return k


BR = 400


def _tc_layer_body(a0, a1, d0, d1, h_ref, wl, blr, wr, o_ref):
    cnt = d0[:, 0:1] + d1[:, 0:1]
    inv = 1.0 / jnp.maximum(cnt, 1.0)
    mean = (a0[...] + a1[...]) * inv
    acc = lax.dot_general(mean, wl[...], (((1,), (1,)), ((), ())),
                          preferred_element_type=jnp.float32,
                          precision=lax.Precision.HIGHEST)
    acc = acc + blr[...]
    acc = acc + lax.dot_general(h_ref[...], wr[...], (((1,), (1,)), ((), ())),
                                preferred_element_type=jnp.float32,
                                precision=lax.Precision.HIGHEST)
    o_ref[...] = jnp.maximum(acc, 0.0)


def _tc_layer(a0, a1, d0, d1, h, Wl, bl, Wr):
    nb = N // BR
    return pl.pallas_call(
        _tc_layer_body,
        grid=(nb,),
        in_specs=[
            pl.BlockSpec((BR, H), lambda i: (i, 0)),
            pl.BlockSpec((BR, H), lambda i: (i, 0)),
            pl.BlockSpec((BR, H), lambda i: (i, 0)),
            pl.BlockSpec((BR, H), lambda i: (i, 0)),
            pl.BlockSpec((BR, H), lambda i: (i, 0)),
            pl.BlockSpec((H, H), lambda i: (0, 0)),
            pl.BlockSpec((1, H), lambda i: (0, 0)),
            pl.BlockSpec((H, H), lambda i: (0, 0)),
        ],
        out_specs=pl.BlockSpec((BR, H), lambda i: (i, 0)),
        out_shape=jax.ShapeDtypeStruct((N, H), jnp.float32),
    )(a0, a1, d0, d1, h, Wl, bl, Wr)


def _tc_final_body(h1, h2, h3, w1, w2, w3, br, o_ref):
    acc = lax.dot_general(h1[...], w1[...], (((1,), (1,)), ((), ())),
                          preferred_element_type=jnp.float32,
                          precision=lax.Precision.HIGHEST)
    acc = acc + lax.dot_general(h2[...], w2[...], (((1,), (1,)), ((), ())),
                                preferred_element_type=jnp.float32,
                                precision=lax.Precision.HIGHEST)
    acc = acc + lax.dot_general(h3[...], w3[...], (((1,), (1,)), ((), ())),
                                preferred_element_type=jnp.float32,
                                precision=lax.Precision.HIGHEST)
    o_ref[...] = acc + br[...]


def _tc_final(h1, h2, h3, w1, w2, w3, fc_b):
    nb = N // BR
    return pl.pallas_call(
        _tc_final_body,
        grid=(nb,),
        in_specs=[
            pl.BlockSpec((BR, H), lambda i: (i, 0)),
            pl.BlockSpec((BR, H), lambda i: (i, 0)),
            pl.BlockSpec((BR, H), lambda i: (i, 0)),
            pl.BlockSpec((OUT, H), lambda i: (0, 0)),
            pl.BlockSpec((OUT, H), lambda i: (0, 0)),
            pl.BlockSpec((OUT, H), lambda i: (0, 0)),
            pl.BlockSpec((1, OUT), lambda i: (0, 0)),
        ],
        out_specs=pl.BlockSpec((BR, OUT), lambda i: (i, 0)),
        out_shape=jax.ShapeDtypeStruct((N, OUT), jnp.float32),
    )(h1, h2, h3, w1, w2, w3, fc_b)


def kernel(x, edge_index, Wl0, bl0, Wr0, Wl1, bl1, Wr1, Wl2, bl2, Wr2, fc_W, fc_b):
    src = edge_index[0]
    dst = edge_index[1]
    e = src.shape[0]
    cpw = -(-e // (NW * CH))
    cpw = -(-cpw // SB) * SB
    e_pad = NW * CH * cpw
    src_p = jnp.concatenate(
        [src, jnp.zeros((e_pad - e,), jnp.int32)]).reshape(-1, CH)
    dst_p = jnp.concatenate(
        [dst, jnp.full((e_pad - e,), N, jnp.int32)]).reshape(-1, CH)

    degs = _sc_degree(dst_p, cpw)
    d0 = degs[0:N]
    d1 = degs[N_PAD:N_PAD + N]

    src_p2 = src_p.reshape(-1, CH2)
    dst_p2 = dst_p.reshape(-1, CH2)
    cpw2 = src_p2.shape[0] // NW
    agg2 = _sc_agg_spmem_build(cpw2)

    h = x
    hs = []
    for (Wl, bl, Wr) in ((Wl0, bl0, Wr0), (Wl1, bl1, Wr1), (Wl2, bl2, Wr2)):
        parts = agg2(h, src_p2, dst_p2)
        a0 = jnp.concatenate([parts[0:NH], parts[NH_PAD:NH_PAD + NH]])
        a1 = jnp.concatenate([parts[2 * NH_PAD:2 * NH_PAD + NH],
                              parts[3 * NH_PAD:3 * NH_PAD + NH]])
        h = _tc_layer(a0, a1, d0, d1, h, Wl, bl.reshape(1, H), Wr)
        hs.append(h)

    return _tc_final(hs[0], hs[1], hs[2],
                     fc_W[:, 0:H], fc_W[:, H:2 * H], fc_W[:, 2 * H:3 * H],
                     fc_b.reshape(1, OUT))
